# baseline (speedup 1.0000x reference)
import functools

import jax
import jax.numpy as jnp
from jax import lax
from jax.experimental import pallas as pl
from jax.experimental.pallas import tpu as pltpu
from jax.experimental.pallas import tpu_sc as plsc

MAX_DEG = 6
N_PER_DEG = 14000
N_ATOMS = (MAX_DEG + 1) * N_PER_DEG
FEAT = 128
BATCH = 64
NPASS = MAX_DEG * (MAX_DEG + 1) // 2

NC = 2
NS = 16
NW = NC * NS
PAD_N = 14080
CHUNK = PAD_N // NW
SUB = 88
NSUB = CHUNK // SUB

BLK = 1000
NBLK = N_PER_DEG // BLK

_PASS_DS = [(d, s) for d in range(1, MAX_DEG + 1) for s in range(d)]
_SC_CHUNKS = [(5, 6), (3, 4), (1, 2)]
_TC_CHUNKS = [(0,), (5, 6), (3, 4), (1, 2)]


def _pass_range(degs):
    ps = [p for p, (d, _) in enumerate(_PASS_DS) if d in degs]
    return ps[0], ps[-1] + 1


def _sc_gather_body(p_lo, p_hi, atoms_hbm, idx_hbm, *rest):
    n_out = len(set(d for d, _ in _PASS_DS[p_lo:p_hi]))
    outs = rest[:n_out]
    idx_f, rows_v, sem_g, sem_s = rest[n_out:]
    d_lo = _PASS_DS[p_lo][0]

    wid = lax.axis_index("s") * NC + lax.axis_index("c")
    base = wid * CHUNK
    npass = p_hi - p_lo

    pltpu.sync_copy(
        idx_hbm.at[pl.ds(wid * (NPASS * CHUNK) + p_lo * CHUNK, npass * CHUNK)],
        idx_f.at[pl.ds(0, npass * CHUNK)])

    def fire(p):
        ph = p % 2
        return [pltpu.async_copy(
            atoms_hbm.at[idx_f.at[pl.ds(p * CHUNK + c * SUB, SUB)]],
            rows_v.at[ph, pl.ds(c * SUB, SUB)], sem_g)
            for c in range(NSUB)]

    def store(p):
        d, s = _PASS_DS[p_lo + p]
        return pltpu.async_copy(rows_v.at[p % 2],
                                outs[d - d_lo].at[s, pl.ds(base, CHUNK)],
                                sem_s)

    gh = fire(0)
    sh = None
    for p in range(npass):
        for h in gh:
            h.wait()
        if sh is not None:
            sh.wait()
        if p + 1 < npass:
            gh = fire(p + 1)
        sh = store(p)
    sh.wait()


@functools.cache
def _make_sc_gather(p_lo, p_hi):
    degs = sorted(set(d for d, _ in _PASS_DS[p_lo:p_hi]))
    npass = p_hi - p_lo
    return pl.kernel(
        functools.partial(_sc_gather_body, p_lo, p_hi),
        out_type=[jax.ShapeDtypeStruct((d, PAD_N, FEAT), jnp.float32)
                  for d in degs],
        mesh=plsc.VectorSubcoreMesh(core_axis_name="c", subcore_axis_name="s",
                                    num_cores=NC, num_subcores=NS),
        scratch_types=[
            pltpu.VMEM((npass * CHUNK,), jnp.int32),
            pltpu.VMEM((2, CHUNK, FEAT), jnp.float32),
            pltpu.SemaphoreType.DMA,
            pltpu.SemaphoreType.DMA,
        ],
    )


def _tc_stage(degs, atoms, gbufs, acc_in, act_in,
              wself, wrel, wgath, bact, bgath, mem_r):
    d0 = degs[0]
    assert all(degs[i] == d0 + i for i in range(len(degs)))
    gdegs = [dd for dd in degs if dd >= 1]

    def body(*refs):
        it = iter(refs)
        atoms_ref = next(it)
        g_refs = [next(it) for _ in gdegs]
        acc_ref = next(it) if acc_in is not None else None
        if act_in is not None:
            next(it)
        wself_r, wrel_r, wgath_r, bact_r, bgath_r, mem_ref = [
            next(it) for _ in range(6)]
        act_out, gath_out = next(it), next(it)

        d = pl.program_id(0)
        j = pl.program_id(1)
        a = atoms_ref[...]

        act = jnp.dot(a, wself_r[0], preferred_element_type=jnp.float32)
        if gdegs:
            ns = jnp.zeros_like(a)
            for i, dd in enumerate(degs):
                if dd >= 1:
                    gi = g_refs[gdegs.index(dd)]
                    ns = jnp.where(d == i, jnp.sum(gi[...], axis=0), ns)
            act = act + jnp.dot(ns, wrel_r[0],
                                preferred_element_type=jnp.float32)
        act_out[...] = act + bact_r[0]

        g = jnp.dot(a, wgath_r[0], preferred_element_type=jnp.float32) \
            + bgath_r[0]
        m = mem_ref[0, 0]
        onehot = (lax.broadcasted_iota(jnp.int32, (BATCH, BLK), 0)
                  == m[None, :]).astype(jnp.float32)
        part = jnp.dot(onehot, g, preferred_element_type=jnp.float32)

        first = (d == 0) & (j == 0)

        @pl.when(first)
        def _():
            gath_out[...] = (part if acc_ref is None
                             else acc_ref[...] + part)

        @pl.when(jnp.logical_not(first))
        def _():
            gath_out[...] += part

    in_specs = [pl.BlockSpec((BLK, FEAT),
                             lambda d, j: ((d + d0) * NBLK + j, 0))]
    for i, dd in enumerate(degs):
        if dd >= 1:
            in_specs.append(pl.BlockSpec(
                (dd, BLK, FEAT),
                lambda d, j, i=i: (0, jnp.where(d == i, j, 0), 0)))
    operands = [atoms] + list(gbufs)
    if acc_in is not None:
        in_specs.append(pl.BlockSpec((BATCH, FEAT), lambda d, j: (0, 0)))
        operands.append(acc_in)
    aliases = {}
    if act_in is not None:
        aliases = {len(operands): 0}
        in_specs.append(pl.BlockSpec(memory_space=pl.ANY))
        operands.append(act_in)
    in_specs += [
        pl.BlockSpec((1, FEAT, FEAT), lambda d, j: (d, 0, 0)),
        pl.BlockSpec((1, FEAT, FEAT), lambda d, j: (d, 0, 0)),
        pl.BlockSpec((1, FEAT, FEAT), lambda d, j: (d, 0, 0)),
        pl.BlockSpec((1, 1, FEAT), lambda d, j: (d, 0, 0)),
        pl.BlockSpec((1, 1, FEAT), lambda d, j: (d, 0, 0)),
        pl.BlockSpec((1, 1, BLK),
                     lambda d, j: ((d + d0) * NBLK + j, 0, 0)),
    ]
    sel = jnp.array(degs, dtype=jnp.int32)
    operands += [wself[sel], wrel[sel], wgath[sel], bact[sel], bgath[sel],
                 mem_r]

    return pl.pallas_call(
        body,
        grid=(len(degs), NBLK),
        in_specs=in_specs,
        out_specs=[
            pl.BlockSpec((BLK, FEAT),
                         lambda d, j: ((d + d0) * NBLK + j, 0)),
            pl.BlockSpec((BATCH, FEAT), lambda d, j: (0, 0)),
        ],
        out_shape=[
            jax.ShapeDtypeStruct((N_ATOMS, FEAT), jnp.float32),
            jax.ShapeDtypeStruct((BATCH, FEAT), jnp.float32),
        ],
        input_output_aliases=aliases,
        compiler_params=pltpu.CompilerParams(
            dimension_semantics=("arbitrary", "arbitrary")),
    )(*operands)


def kernel(atoms, deg_slice, membership, deg_adj_1, deg_adj_2, deg_adj_3,
           deg_adj_4, deg_adj_5, deg_adj_6, W_stack, b_stack):
    adjs = [deg_adj_1, deg_adj_2, deg_adj_3, deg_adj_4, deg_adj_5, deg_adj_6]
    idx_rows = jnp.concatenate([a.T for a in adjs], axis=0)
    idx_rows = jnp.pad(idx_rows, ((0, 0), (0, PAD_N - N_PER_DEG)))
    idx_tiles = idx_rows.reshape(NPASS, NW, CHUNK).transpose(1, 0, 2).reshape(-1)

    gb = {}
    for degs in _SC_CHUNKS:
        p_lo, p_hi = _pass_range(degs)
        outs = _make_sc_gather(p_lo, p_hi)(atoms, idx_tiles)
        for dd, o in zip(sorted(degs), outs):
            gb[dd] = o

    i_self = jnp.array([12, 1, 3, 5, 7, 9, 11], dtype=jnp.int32)
    i_gath = jnp.array([19, 13, 14, 15, 16, 17, 18], dtype=jnp.int32)
    i_rel = jnp.array([0, 0, 2, 4, 6, 8, 10], dtype=jnp.int32)
    wself = W_stack[i_self]
    wgath = W_stack[i_gath]
    wrel = W_stack[i_rel].at[0].set(0.0)
    bact = (b_stack[i_self] + b_stack[i_rel].at[0].set(0.0)).reshape(
        MAX_DEG + 1, 1, FEAT)
    bgath = b_stack[i_gath].reshape(MAX_DEG + 1, 1, FEAT)
    mem_r = membership.reshape(N_ATOMS // BLK, 1, BLK)

    act, acc = None, None
    for degs in _TC_CHUNKS:
        act, acc = _tc_stage(degs, atoms, [gb[dd] for dd in degs if dd >= 1],
                             acc, act, wself, wrel, wgath, bact, bgath, mem_r)
    return act, acc
